# baseline (speedup 1.0000x reference)
import functools
import jax
import jax.numpy as jnp
from jax import lax
from jax.experimental import pallas as pl
from jax.experimental.pallas import tpu as pltpu
from jax.experimental.pallas import tpu_sc as plsc

N = 10000
E = 320000
D_IN = 128
D_H = 256
CLASSES = 10
G = 64

NS = 16
GPT = 160
E_PAD = NS * GPT * 128
N_PAD = 10112


def _make_segsum_sc(split_edges):
    mesh = plsc.VectorSubcoreMesh(core_axis_name="c", subcore_axis_name="s")
    gpw = (GPT * 2) // 2 if split_edges else GPT * 2
    C = 16
    NB = 5

    @functools.partial(
        pl.kernel,
        mesh=mesh,
        out_type=jax.ShapeDtypeStruct((2, N_PAD, 128), jnp.float32),
        scratch_types=[
            pltpu.VMEM((C, 64), jnp.int32),
            pltpu.VMEM((C, 64), jnp.int32),
            pltpu.VMEM((NB, 64, 128), jnp.float32),
            pltpu.VMEM_SHARED((N_PAD, 128), jnp.float32),
            pltpu.SemaphoreType.DMA,
            pltpu.SemaphoreType.DMA,
            pltpu.SemaphoreType.DMA,
            pltpu.SemaphoreType.DMA,
            pltpu.SemaphoreType.DMA,
            pltpu.SemaphoreType.DMA,
        ],
    )
    def segsum(tables, src2, dst2, zeros, out, src_v, dst_v, rows_v, acc,
               sem_g0, sem_g1, sem_g2, sem_g3, sem_g4, sem_s):
        cid = lax.axis_index("c")
        sid = lax.axis_index("s")

        z0 = sid * (N_PAD // NS)
        pltpu.sync_copy(zeros, acc.at[pl.ds(z0, N_PAD // NS)])

        if split_edges:
            g0 = (cid * NS + sid) * gpw
            table = tables.at[0]
        else:
            g0 = sid * gpw
            table = tables.at[cid]
        plsc.subcore_barrier()

        sem_g = (sem_g0, sem_g1, sem_g2, sem_g3, sem_g4)

        def gather(j):
            return pltpu.async_copy(table.at[src_v.at[j]],
                                    rows_v.at[j % NB], sem_g[j % NB])

        def chunk_body(ci, carry):
            g0c = g0 + ci * C
            pltpu.sync_copy(src2.at[pl.ds(g0c, C)], src_v)
            pltpu.sync_copy(dst2.at[pl.ds(g0c, C)], dst_v)

            hg = [gather(j) for j in range(NB - 1)]
            hs = None
            for j in range(C):
                hg.pop(0).wait()
                if hs is not None:
                    hs.wait()
                hs = pltpu.async_copy(rows_v.at[j % NB],
                                      acc.at[dst_v.at[j]], sem_s, add=True)
                if j + NB - 1 < C:
                    hg.append(gather(j + NB - 1))
            hs.wait()
            return carry

        lax.fori_loop(0, gpw // C, chunk_body, 0)
        plsc.subcore_barrier()

        r0 = sid * (N_PAD // NS)
        pltpu.sync_copy(acc.at[pl.ds(r0, N_PAD // NS)],
                        out.at[cid].at[pl.ds(r0, N_PAD // NS)])

    return segsum

R1 = 400
R2 = 400


def _l1_body(eps_ref, x_ref, agg_ref, w_ref, m_ref, b_ref, z_ref):
    wm = w_ref[...] * m_ref[...]
    agg = agg_ref[0] + agg_ref[1]
    h = (1.0 + eps_ref[0, 0]) * x_ref[...] + agg
    out = lax.dot_general(h, wm, (((1,), (1,)), ((), ())),
                          preferred_element_type=jnp.float32)
    out = jnp.maximum(out + b_ref[...], 0.0)
    z_ref[0] = out[:, :128]
    z_ref[1] = out[:, 128:]


def _layer1_tc(x, agg1s, W1, mask1, b1, eps1):
    grid = (N // R1,)
    return pl.pallas_call(
        _l1_body,
        grid=grid,
        in_specs=[
            pl.BlockSpec((1, 1), lambda i: (0, 0)),
            pl.BlockSpec((R1, D_IN), lambda i: (i, 0)),
            pl.BlockSpec((2, R1, 128), lambda i: (0, i, 0)),
            pl.BlockSpec((D_H, D_IN), lambda i: (0, 0)),
            pl.BlockSpec((D_H, D_IN), lambda i: (0, 0)),
            pl.BlockSpec((1, D_H), lambda i: (0, 0)),
        ],
        out_specs=pl.BlockSpec((2, R1, 128), lambda i: (0, i, 0)),
        out_shape=jax.ShapeDtypeStruct((2, N, 128), jnp.float32),
    )(eps1.reshape(1, 1), x, agg1s, W1, mask1, b1.reshape(1, D_H))


def _l2_body(eps_ref, z_ref, a_ref, w_ref, m_ref, b_ref, batch_ref,
             wp_ref, bp_ref, out_ref, sums, counts):
    i = pl.program_id(0)

    @pl.when(i == 0)
    def _():
        sums[...] = jnp.zeros_like(sums)
        counts[...] = jnp.zeros_like(counts)

    wm = w_ref[...] * m_ref[...]
    z1 = jnp.concatenate([z_ref[0], z_ref[1]], axis=1)
    agg = jnp.concatenate([a_ref[0], a_ref[1]], axis=1)
    h = (1.0 + eps_ref[0, 0]) * z1 + agg
    z2 = lax.dot_general(h, wm, (((1,), (1,)), ((), ())),
                         preferred_element_type=jnp.float32)
    z2 = jnp.maximum(z2 + b_ref[...], 0.0)

    bvec = batch_ref[0, 0, :]
    gids = lax.broadcasted_iota(jnp.int32, (G, R2), 0)
    P = (gids == bvec[None, :]).astype(jnp.float32)
    sums[...] += lax.dot_general(P, z2, (((1,), (0,)), ((), ())),
                                 preferred_element_type=jnp.float32)
    counts[...] += jnp.sum(P, axis=1, keepdims=True)

    @pl.when(i == pl.num_programs(0) - 1)
    def _():
        gz = sums[...] / jnp.maximum(counts[...], 1.0)
        logit = lax.dot_general(gz, wp_ref[...], (((1,), (1,)), ((), ())),
                                preferred_element_type=jnp.float32)
        out_ref[...] = jax.nn.sigmoid(logit + bp_ref[...])


def _layer2_tc(z1s, agg2s, W2, mask2, b2, eps2, batch, Wp, bp):
    grid = (N // R2,)
    batch3 = batch.reshape(N // R2, 1, R2)
    return pl.pallas_call(
        _l2_body,
        grid=grid,
        in_specs=[
            pl.BlockSpec((1, 1), lambda i: (0, 0)),
            pl.BlockSpec((2, R2, 128), lambda i: (0, i, 0)),
            pl.BlockSpec((2, R2, 128), lambda i: (0, i, 0)),
            pl.BlockSpec((D_H, D_H), lambda i: (0, 0)),
            pl.BlockSpec((D_H, D_H), lambda i: (0, 0)),
            pl.BlockSpec((1, D_H), lambda i: (0, 0)),
            pl.BlockSpec((1, 1, R2), lambda i: (i, 0, 0)),
            pl.BlockSpec((CLASSES, D_H), lambda i: (0, 0)),
            pl.BlockSpec((1, CLASSES), lambda i: (0, 0)),
        ],
        out_specs=pl.BlockSpec((G, CLASSES), lambda i: (0, 0)),
        out_shape=jax.ShapeDtypeStruct((G, CLASSES), jnp.float32),
        scratch_shapes=[
            pltpu.VMEM((G, D_H), jnp.float32),
            pltpu.VMEM((G, 1), jnp.float32),
        ],
    )(eps2.reshape(1, 1), z1s, agg2s, W2, mask2, b2.reshape(1, D_H),
      batch3, Wp, bp.reshape(1, CLASSES))


_segsum_l1 = _make_segsum_sc(split_edges=True)
_segsum_l2 = _make_segsum_sc(split_edges=False)


def kernel(x, edge_index, batch, weights_mask1, weights_mask2,
           W1, b1, eps1, W2, b2, eps2, Wp, bp):
    src = edge_index[0]
    dst = edge_index[1]
    pad = E_PAD - E
    pad_i = jnp.arange(pad, dtype=jnp.int32)
    src2 = jnp.concatenate(
        [src, pad_i % 256]).reshape(NS * GPT * 2, 64)
    dst2 = jnp.concatenate(
        [dst, N + pad_i % (N_PAD - N)]).reshape(NS * GPT * 2, 64)
    zeros128 = jnp.zeros((N_PAD // NS, 128), jnp.float32)

    agg1s = _segsum_l1(x.reshape(1, N, D_IN), src2, dst2, zeros128)
    z1s = _layer1_tc(x, agg1s, W1, weights_mask1, b1, eps1)
    agg2s = _segsum_l2(z1s, src2, dst2, zeros128)
    return _layer2_tc(z1s, agg2s, W2, weights_mask2, b2, eps2,
                      batch, Wp, bp)


# speedup vs baseline: 1.0187x; 1.0187x over previous
import functools
import jax
import jax.numpy as jnp
from jax import lax
from jax.experimental import pallas as pl
from jax.experimental.pallas import tpu as pltpu
from jax.experimental.pallas import tpu_sc as plsc

N = 10000
E = 320000
D_IN = 128
D_H = 256
CLASSES = 10
G = 64

NS = 16
GPT = 160
E_PAD = NS * GPT * 128
N_PAD = 10112


def _make_segsum_sc(split_edges):
    mesh = plsc.VectorSubcoreMesh(core_axis_name="c", subcore_axis_name="s")
    gpw = (GPT * 2) // 2 if split_edges else GPT * 2
    C = 16
    NB = 4

    @functools.partial(
        pl.kernel,
        mesh=mesh,
        out_type=jax.ShapeDtypeStruct((2, N_PAD, 128), jnp.float32),
        scratch_types=[
            pltpu.VMEM((C, 64), jnp.int32),
            pltpu.VMEM((C, 64), jnp.int32),
            pltpu.VMEM((NB, 64, 128), jnp.float32),
            pltpu.VMEM_SHARED((N_PAD, 128), jnp.float32),
            pltpu.SemaphoreType.DMA,
            pltpu.SemaphoreType.DMA,
            pltpu.SemaphoreType.DMA,
            pltpu.SemaphoreType.DMA,
            pltpu.SemaphoreType.DMA,
        ],
    )
    def segsum(tables, src2, dst2, zeros, out, src_v, dst_v, rows_v, acc,
               sem_g0, sem_g1, sem_g2, sem_g3, sem_s):
        cid = lax.axis_index("c")
        sid = lax.axis_index("s")

        z0 = sid * (N_PAD // NS)
        pltpu.sync_copy(zeros, acc.at[pl.ds(z0, N_PAD // NS)])

        if split_edges:
            g0 = (cid * NS + sid) * gpw
            table = tables.at[0]
        else:
            g0 = sid * gpw
            table = tables.at[cid]
        plsc.subcore_barrier()

        sem_g = (sem_g0, sem_g1, sem_g2, sem_g3)

        def gather(j):
            return pltpu.async_copy(table.at[src_v.at[j]],
                                    rows_v.at[j % NB], sem_g[j % NB])

        def chunk_body(ci, carry):
            g0c = g0 + ci * C
            pltpu.sync_copy(src2.at[pl.ds(g0c, C)], src_v)
            pltpu.sync_copy(dst2.at[pl.ds(g0c, C)], dst_v)

            hg = [gather(j) for j in range(NB - 1)]
            hs = None
            for j in range(C):
                hg.pop(0).wait()
                if hs is not None:
                    hs.wait()
                hs = pltpu.async_copy(rows_v.at[j % NB],
                                      acc.at[dst_v.at[j]], sem_s, add=True)
                if j + NB - 1 < C:
                    hg.append(gather(j + NB - 1))
            hs.wait()
            return carry

        lax.fori_loop(0, gpw // C, chunk_body, 0)
        plsc.subcore_barrier()

        r0 = sid * (N_PAD // NS)
        pltpu.sync_copy(acc.at[pl.ds(r0, N_PAD // NS)],
                        out.at[cid].at[pl.ds(r0, N_PAD // NS)])

    return segsum

R1 = 400
R2 = 400


def _l1_body(eps_ref, x_ref, agg_ref, w_ref, m_ref, b_ref, z_ref):
    wm = w_ref[...] * m_ref[...]
    agg = agg_ref[0] + agg_ref[1]
    h = (1.0 + eps_ref[0, 0]) * x_ref[...] + agg
    out = lax.dot_general(h, wm, (((1,), (1,)), ((), ())),
                          preferred_element_type=jnp.float32)
    out = jnp.maximum(out + b_ref[...], 0.0)
    z_ref[0] = out[:, :128]
    z_ref[1] = out[:, 128:]


def _layer1_tc(x, agg1s, W1, mask1, b1, eps1):
    grid = (N // R1,)
    return pl.pallas_call(
        _l1_body,
        grid=grid,
        in_specs=[
            pl.BlockSpec((1, 1), lambda i: (0, 0)),
            pl.BlockSpec((R1, D_IN), lambda i: (i, 0)),
            pl.BlockSpec((2, R1, 128), lambda i: (0, i, 0)),
            pl.BlockSpec((D_H, D_IN), lambda i: (0, 0)),
            pl.BlockSpec((D_H, D_IN), lambda i: (0, 0)),
            pl.BlockSpec((1, D_H), lambda i: (0, 0)),
        ],
        out_specs=pl.BlockSpec((2, R1, 128), lambda i: (0, i, 0)),
        out_shape=jax.ShapeDtypeStruct((2, N, 128), jnp.float32),
    )(eps1.reshape(1, 1), x, agg1s, W1, mask1, b1.reshape(1, D_H))


def _l2_body(eps_ref, z_ref, a_ref, w_ref, m_ref, b_ref, batch_ref,
             wp_ref, bp_ref, out_ref, sums, counts):
    i = pl.program_id(0)

    @pl.when(i == 0)
    def _():
        sums[...] = jnp.zeros_like(sums)
        counts[...] = jnp.zeros_like(counts)

    wm = w_ref[...] * m_ref[...]
    z1 = jnp.concatenate([z_ref[0], z_ref[1]], axis=1)
    agg = jnp.concatenate([a_ref[0], a_ref[1]], axis=1)
    h = (1.0 + eps_ref[0, 0]) * z1 + agg
    z2 = lax.dot_general(h, wm, (((1,), (1,)), ((), ())),
                         preferred_element_type=jnp.float32)
    z2 = jnp.maximum(z2 + b_ref[...], 0.0)

    bvec = batch_ref[0, 0, :]
    gids = lax.broadcasted_iota(jnp.int32, (G, R2), 0)
    P = (gids == bvec[None, :]).astype(jnp.float32)
    sums[...] += lax.dot_general(P, z2, (((1,), (0,)), ((), ())),
                                 preferred_element_type=jnp.float32)
    counts[...] += jnp.sum(P, axis=1, keepdims=True)

    @pl.when(i == pl.num_programs(0) - 1)
    def _():
        gz = sums[...] / jnp.maximum(counts[...], 1.0)
        logit = lax.dot_general(gz, wp_ref[...], (((1,), (1,)), ((), ())),
                                preferred_element_type=jnp.float32)
        out_ref[...] = jax.nn.sigmoid(logit + bp_ref[...])


def _layer2_tc(z1s, agg2s, W2, mask2, b2, eps2, batch, Wp, bp):
    grid = (N // R2,)
    batch3 = batch.reshape(N // R2, 1, R2)
    return pl.pallas_call(
        _l2_body,
        grid=grid,
        in_specs=[
            pl.BlockSpec((1, 1), lambda i: (0, 0)),
            pl.BlockSpec((2, R2, 128), lambda i: (0, i, 0)),
            pl.BlockSpec((2, R2, 128), lambda i: (0, i, 0)),
            pl.BlockSpec((D_H, D_H), lambda i: (0, 0)),
            pl.BlockSpec((D_H, D_H), lambda i: (0, 0)),
            pl.BlockSpec((1, D_H), lambda i: (0, 0)),
            pl.BlockSpec((1, 1, R2), lambda i: (i, 0, 0)),
            pl.BlockSpec((CLASSES, D_H), lambda i: (0, 0)),
            pl.BlockSpec((1, CLASSES), lambda i: (0, 0)),
        ],
        out_specs=pl.BlockSpec((G, CLASSES), lambda i: (0, 0)),
        out_shape=jax.ShapeDtypeStruct((G, CLASSES), jnp.float32),
        scratch_shapes=[
            pltpu.VMEM((G, D_H), jnp.float32),
            pltpu.VMEM((G, 1), jnp.float32),
        ],
    )(eps2.reshape(1, 1), z1s, agg2s, W2, mask2, b2.reshape(1, D_H),
      batch3, Wp, bp.reshape(1, CLASSES))


_segsum_l1 = _make_segsum_sc(split_edges=True)
_segsum_l2 = _make_segsum_sc(split_edges=False)


def kernel(x, edge_index, batch, weights_mask1, weights_mask2,
           W1, b1, eps1, W2, b2, eps2, Wp, bp):
    src = edge_index[0]
    dst = edge_index[1]
    pad = E_PAD - E
    pad_i = jnp.arange(pad, dtype=jnp.int32)
    src2 = jnp.concatenate(
        [src, pad_i % 256]).reshape(NS * GPT * 2, 64)
    dst2 = jnp.concatenate(
        [dst, N + pad_i % (N_PAD - N)]).reshape(NS * GPT * 2, 64)
    zeros128 = jnp.zeros((N_PAD // NS, 128), jnp.float32)

    agg1s = _segsum_l1(x.reshape(1, N, D_IN), src2, dst2, zeros128)
    z1s = _layer1_tc(x, agg1s, W1, weights_mask1, b1, eps1)
    agg2s = _segsum_l2(z1s, src2, dst2, zeros128)
    return _layer2_tc(z1s, agg2s, W2, weights_mask2, b2, eps2,
                      batch, Wp, bp)
